# baseline (speedup 1.0000x reference)
import jax
import jax.numpy as jnp
from jax import lax
from jax.experimental import pallas as pl
from jax.experimental.pallas import tpu as pltpu
from jax.experimental.pallas import tpu_sc as plsc

DIM = 64
N_IDX = 16384
TILE_R = 8
NC, NS = 2, 16
NW = NC * NS
B_PER_W = N_IDX // NW


def _gather_body(idx_hbm, table_hbm, out_hbm, idx_v, rows_v, sem):
    wid = lax.axis_index("s") * NC + lax.axis_index("c")
    base = wid * B_PER_W
    pltpu.sync_copy(idx_hbm.at[pl.ds(base, B_PER_W)],
                    idx_v.at[pl.ds(0, B_PER_W)])

    def body(n, _):
        r = idx_v[pl.ds(n, 16)][0]
        pltpu.make_async_copy(
            table_hbm.at[r >> 3, r & (TILE_R - 1)],
            rows_v.at[n // TILE_R, n % TILE_R],
            sem,
        ).start()
        return 0

    lax.fori_loop(0, B_PER_W, body, 0)
    pltpu.make_async_copy(table_hbm.at[pl.ds(0, B_PER_W // TILE_R)],
                          rows_v, sem).wait()
    pltpu.sync_copy(rows_v,
                    out_hbm.at[pl.ds(base // TILE_R, B_PER_W // TILE_R)])


@jax.jit
def _gather(indics, table3):
    mesh = plsc.VectorSubcoreMesh(core_axis_name="c", subcore_axis_name="s")
    return pl.kernel(
        _gather_body,
        out_type=jax.ShapeDtypeStruct((N_IDX // TILE_R, TILE_R, DIM),
                                      jnp.float32),
        mesh=mesh,
        scratch_types=[
            pltpu.VMEM((B_PER_W + 16,), jnp.int32),
            pltpu.VMEM((B_PER_W // TILE_R, TILE_R, DIM), jnp.float32),
            pltpu.SemaphoreType.DMA,
        ],
    )(indics, table3)


def kernel(indics, offset, weight):
    del offset
    table3 = weight.reshape(weight.shape[0] // TILE_R, TILE_R, DIM)
    out3 = _gather(indics, table3)
    return out3.reshape(N_IDX, DIM)
